# baseline (speedup 1.0000x reference)
import jax
import jax.numpy as jnp
from jax import lax
from jax.experimental import pallas as pl
from jax.experimental.pallas import tpu as pltpu
from jax.experimental.pallas import tpu_sc as plsc

N = 10000
E = 320000
D = 128

NC, NS = 2, 16
NW = NC * NS
NPAD = 10240
NBR = NPAD // 128

CH = 80
CPT = 128
EPAD = NW * CPT * CH
NBUF = 4
PASS_CH = 32
NPASS = CPT // PASS_CH
ROWS_PER_TILE = NPAD // NS

_MESH = plsc.VectorSubcoreMesh(
    core_axis_name="c", subcore_axis_name="s", num_cores=NC, num_subcores=NS)
_SC_PARAMS = pltpu.CompilerParams(needs_layout_passes=False)


def _zero_rows(ref, nrows, ncols):
    z16 = jnp.zeros((16,), jnp.float32)

    def body(r, _):
        for k in range(ncols // 16):
            ref[r, pl.ds(k * 16, 16)] = z16
        return 0
    lax.fori_loop(0, nrows, body, 0)


def _sc_degrees(row_hbm, col_hbm, zflat_hbm, deg_hbm, rowv, colv, h):
    c = lax.axis_index("c")
    s = lax.axis_index("s")
    wid = c * NS + s
    pltpu.sync_copy(row_hbm.at[pl.ds(wid * CPT, CPT)], rowv)
    pltpu.sync_copy(col_hbm.at[pl.ds(wid * CPT, CPT)], colv)
    pltpu.sync_copy(zflat_hbm, h)

    ones16 = jnp.ones((16,), jnp.float32)

    def hloop(j, _):
        for k in range(CH // 16):
            cv = colv[j, pl.ds(k * 16, 16)]
            plsc.addupdate_scatter(h, [cv], ones16)
            rv = rowv[j, pl.ds(k * 16, 16)]
            plsc.addupdate_scatter(h, [rv + NPAD], ones16)
        return 0
    lax.fori_loop(0, CPT, hloop, 0)

    pltpu.sync_copy(h.at[pl.ds(0, NPAD)], deg_hbm.at[wid, 0])
    pltpu.sync_copy(h.at[pl.ds(NPAD, NPAD)], deg_hbm.at[wid, 1])


_degrees = pl.kernel(
    _sc_degrees,
    out_type=jax.ShapeDtypeStruct((NW, 2, NPAD), jnp.float32),
    mesh=_MESH,
    compiler_params=_SC_PARAMS,
    scratch_types=[
        pltpu.VMEM((CPT, CH), jnp.int32),
        pltpu.VMEM((CPT, CH), jnp.int32),
        pltpu.VMEM((2 * NPAD,), jnp.float32),
    ],
)


BLK_P = 256


def _row_scale_col(deg_blk):
    d = jnp.sum(deg_blk, axis=0, keepdims=True)
    inv = jnp.where(d > 0, lax.rsqrt(d), 0.0)
    return jnp.transpose(inv)


def _tc_matmul(x_ref, w_ref, y_ref):
    y_ref[...] = lax.dot_general(x_ref[...], w_ref[...],
                                 (((1,), (1,)), ((), ())),
                                 preferred_element_type=jnp.float32)


_matmul = pl.pallas_call(
    _tc_matmul,
    grid=(NPAD // BLK_P,),
    in_specs=[
        pl.BlockSpec((BLK_P, D), lambda i: (i, 0)),
        pl.BlockSpec((D, D), lambda i: (0, 0)),
    ],
    out_specs=pl.BlockSpec((BLK_P, D), lambda i: (i, 0)),
    out_shape=jax.ShapeDtypeStruct((NPAD, D), jnp.float32),
)


def _tc_scale(y_ref, deg_ref, z_ref):
    inv = _row_scale_col(deg_ref[0, :, 0, :])
    z_ref[...] = y_ref[...] * inv


_scale = pl.pallas_call(
    _tc_scale,
    grid=(NPAD // BLK_P,),
    in_specs=[
        pl.BlockSpec((BLK_P, D), lambda i: (i, 0)),
        pl.BlockSpec((1, NW, 2, BLK_P), lambda i: (0, 0, 0, i)),
    ],
    out_specs=pl.BlockSpec((BLK_P, D), lambda i: (i, 0)),
    out_shape=jax.ShapeDtypeStruct((NPAD, D), jnp.float32),
)


def _sc_scatter(row_hbm, col_hbm, z_hbm, zrows_hbm, acc_hbm, rowv, colv,
                zbuf, accs, gs0, gs1, gs2, gs3, ss0, ss1, ss2, ss3):
    gs = (gs0, gs1, gs2, gs3)
    ss = (ss0, ss1, ss2, ss3)
    c = lax.axis_index("c")
    s = lax.axis_index("s")
    wid = c * NS + s

    pltpu.sync_copy(zrows_hbm, accs.at[pl.ds(s * ROWS_PER_TILE, ROWS_PER_TILE)])
    plsc.subcore_barrier()

    def gather(j, b):
        return pltpu.async_copy(z_hbm.at[colv.at[j]], zbuf.at[b], gs[b])

    def wait_gather(j, b):
        pltpu.make_async_copy(z_hbm.at[colv.at[j]], zbuf.at[b], gs[b]).wait()

    def scatter(j, b):
        return pltpu.async_copy(zbuf.at[b], accs.at[rowv.at[j]], ss[b],
                                add=True)

    def wait_scatter(j, b):
        pltpu.make_async_copy(zbuf.at[b], accs.at[rowv.at[j]], ss[b]).wait()

    for p in range(NPASS):
        base = wid * CPT + p * PASS_CH
        pltpu.sync_copy(row_hbm.at[pl.ds(base, PASS_CH)], rowv)
        pltpu.sync_copy(col_hbm.at[pl.ds(base, PASS_CH)], colv)
        for b in range(NBUF):
            gather(b, b)

        def grp(q, _):
            j0 = q * NBUF
            for b in range(NBUF):
                wait_gather(j0 + b, b)
                scatter(j0 + b, b)
            for b in range(NBUF):
                wait_scatter(j0 + b, b)
                gather(j0 + NBUF + b, b)
            return 0
        lax.fori_loop(0, PASS_CH // NBUF - 1, grp, 0)

        j0 = PASS_CH - NBUF
        for b in range(NBUF):
            wait_gather(j0 + b, b)
            scatter(j0 + b, b)
        for b in range(NBUF):
            wait_scatter(j0 + b, b)

    plsc.subcore_barrier()
    pltpu.sync_copy(accs.at[pl.ds(s * ROWS_PER_TILE, ROWS_PER_TILE)],
                    acc_hbm.at[c, pl.ds(s * ROWS_PER_TILE, ROWS_PER_TILE)])


_scatter = pl.kernel(
    _sc_scatter,
    out_type=jax.ShapeDtypeStruct((NC, NPAD, D), jnp.float32),
    mesh=_MESH,
    compiler_params=_SC_PARAMS,
    scratch_types=[
        pltpu.VMEM((PASS_CH, CH), jnp.int32),
        pltpu.VMEM((PASS_CH, CH), jnp.int32),
        pltpu.VMEM((NBUF, CH, D), jnp.float32),
        pltpu.VMEM_SHARED((NPAD, D), jnp.float32),
        pltpu.SemaphoreType.DMA,
        pltpu.SemaphoreType.DMA,
        pltpu.SemaphoreType.DMA,
        pltpu.SemaphoreType.DMA,
        pltpu.SemaphoreType.DMA,
        pltpu.SemaphoreType.DMA,
        pltpu.SemaphoreType.DMA,
        pltpu.SemaphoreType.DMA,
    ],
)


BLK_F = 512


def _tc_finalize(acc_ref, deg_ref, b_ref, o_ref):
    inv = _row_scale_col(deg_ref[0, :, 1, :])
    o_ref[...] = inv * (acc_ref[0] + acc_ref[1]) + b_ref[...]


_finalize = pl.pallas_call(
    _tc_finalize,
    grid=(NPAD // BLK_F,),
    in_specs=[
        pl.BlockSpec((NC, BLK_F, D), lambda i: (0, i, 0)),
        pl.BlockSpec((1, NW, 2, BLK_F), lambda i: (0, 0, 0, i)),
        pl.BlockSpec((1, D), lambda i: (0, 0)),
    ],
    out_specs=pl.BlockSpec((BLK_F, D), lambda i: (i, 0)),
    out_shape=jax.ShapeDtypeStruct((N, D), jnp.float32),
)


def kernel(x, edge_index, W_src, b_src, W_dst, b_dst):
    del W_dst, b_dst
    pad = EPAD - E
    sent = N + (jnp.arange(pad, dtype=jnp.int32) % (NPAD - N))
    row_p = jnp.concatenate([edge_index[0], sent]).reshape(NW * CPT, CH)
    col_p = jnp.concatenate([edge_index[1], sent]).reshape(NW * CPT, CH)
    x_p = jnp.pad(x, ((0, NPAD - N), (0, 0)))

    zflat = jnp.zeros((2 * NPAD,), jnp.float32)
    zrows = jnp.zeros((ROWS_PER_TILE, D), jnp.float32)

    deg = _degrees(row_p, col_p, zflat)
    deg4 = deg[None]
    y = _matmul(x_p, W_src)
    z = _scale(y, deg4)
    acc = _scatter(row_p, col_p, z, zrows)
    return _finalize(acc, deg4, b_src.reshape(1, D))


# speedup vs baseline: 1.1103x; 1.1103x over previous
import jax
import jax.numpy as jnp
from jax import lax
from jax.experimental import pallas as pl
from jax.experimental.pallas import tpu as pltpu
from jax.experimental.pallas import tpu_sc as plsc

N = 10000
E = 320000
D = 128

NC, NS = 2, 16
NW = NC * NS
NPAD = 10240
NBR = NPAD // 128

CH = 80
CPT = 128
EPAD = NW * CPT * CH
NBUF = 4
PASS_CH = 32
NPASS = CPT // PASS_CH
ROWS_PER_TILE = NPAD // NS

_MESH = plsc.VectorSubcoreMesh(
    core_axis_name="c", subcore_axis_name="s", num_cores=NC, num_subcores=NS)
_SC_PARAMS = pltpu.CompilerParams(needs_layout_passes=False)


def _zero_rows(ref, nrows, ncols):
    z16 = jnp.zeros((16,), jnp.float32)

    def body(r, _):
        for k in range(ncols // 16):
            ref[r, pl.ds(k * 16, 16)] = z16
        return 0
    lax.fori_loop(0, nrows, body, 0)


def _sc_degrees(row_hbm, col_hbm, zflat_hbm, deg_hbm, rowv, colv, h):
    c = lax.axis_index("c")
    s = lax.axis_index("s")
    wid = c * NS + s
    pltpu.sync_copy(row_hbm.at[pl.ds(wid * CPT, CPT)], rowv)
    pltpu.sync_copy(col_hbm.at[pl.ds(wid * CPT, CPT)], colv)
    pltpu.sync_copy(zflat_hbm, h)

    ones16 = jnp.ones((16,), jnp.float32)

    def hloop(j, _):
        for k in range(CH // 16):
            cv = colv[j, pl.ds(k * 16, 16)]
            plsc.addupdate_scatter(h, [cv], ones16)
            rv = rowv[j, pl.ds(k * 16, 16)]
            plsc.addupdate_scatter(h, [rv + NPAD], ones16)
        return 0
    lax.fori_loop(0, CPT, hloop, 0)

    pltpu.sync_copy(h.at[pl.ds(0, NPAD)], deg_hbm.at[wid, 0])
    pltpu.sync_copy(h.at[pl.ds(NPAD, NPAD)], deg_hbm.at[wid, 1])


_degrees = pl.kernel(
    _sc_degrees,
    out_type=jax.ShapeDtypeStruct((NW, 2, NPAD), jnp.float32),
    mesh=_MESH,
    compiler_params=_SC_PARAMS,
    scratch_types=[
        pltpu.VMEM((CPT, CH), jnp.int32),
        pltpu.VMEM((CPT, CH), jnp.int32),
        pltpu.VMEM((2 * NPAD,), jnp.float32),
    ],
)


BLK_P = 512


def _row_scale_col(deg_blk):
    d = jnp.sum(deg_blk, axis=0, keepdims=True)
    inv = jnp.where(d > 0, lax.rsqrt(d), 0.0)
    return jnp.transpose(inv)


def _tc_project(x_ref, w_ref, deg_ref, z_ref):
    inv = _row_scale_col(deg_ref[0, :, 0, :])
    y = lax.dot_general(x_ref[...], w_ref[...], (((1,), (1,)), ((), ())),
                        preferred_element_type=jnp.float32)
    z_ref[...] = y * inv


_project = pl.pallas_call(
    _tc_project,
    grid=(NPAD // BLK_P,),
    in_specs=[
        pl.BlockSpec((BLK_P, D), lambda i: (i, 0)),
        pl.BlockSpec((D, D), lambda i: (0, 0)),
        pl.BlockSpec((1, NW, 2, BLK_P), lambda i: (0, 0, 0, i)),
    ],
    out_specs=pl.BlockSpec((BLK_P, D), lambda i: (i, 0)),
    out_shape=jax.ShapeDtypeStruct((NPAD, D), jnp.float32),
)


def _sc_scatter(row_hbm, col_hbm, z_hbm, zrows_hbm, acc_hbm, rowv, colv,
                zbuf, accs, gs0, gs1, gs2, gs3, ss0, ss1, ss2, ss3):
    gs = (gs0, gs1, gs2, gs3)
    ss = (ss0, ss1, ss2, ss3)
    c = lax.axis_index("c")
    s = lax.axis_index("s")
    wid = c * NS + s

    pltpu.sync_copy(zrows_hbm, accs.at[pl.ds(s * ROWS_PER_TILE, ROWS_PER_TILE)])
    plsc.subcore_barrier()

    def gather(j, b):
        return pltpu.async_copy(z_hbm.at[colv.at[j]], zbuf.at[b], gs[b])

    def wait_gather(j, b):
        pltpu.make_async_copy(z_hbm.at[colv.at[j]], zbuf.at[b], gs[b]).wait()

    def scatter(j, b):
        return pltpu.async_copy(zbuf.at[b], accs.at[rowv.at[j]], ss[b],
                                add=True)

    def wait_scatter(j, b):
        pltpu.make_async_copy(zbuf.at[b], accs.at[rowv.at[j]], ss[b]).wait()

    for p in range(NPASS):
        base = wid * CPT + p * PASS_CH
        pltpu.sync_copy(row_hbm.at[pl.ds(base, PASS_CH)], rowv)
        pltpu.sync_copy(col_hbm.at[pl.ds(base, PASS_CH)], colv)
        for b in range(NBUF):
            gather(b, b)

        def grp(q, _):
            j0 = q * NBUF
            for b in range(NBUF):
                wait_gather(j0 + b, b)
                scatter(j0 + b, b)
            for b in range(NBUF):
                wait_scatter(j0 + b, b)
                gather(j0 + NBUF + b, b)
            return 0
        lax.fori_loop(0, PASS_CH // NBUF - 1, grp, 0)

        j0 = PASS_CH - NBUF
        for b in range(NBUF):
            wait_gather(j0 + b, b)
            scatter(j0 + b, b)
        for b in range(NBUF):
            wait_scatter(j0 + b, b)

    plsc.subcore_barrier()
    pltpu.sync_copy(accs.at[pl.ds(s * ROWS_PER_TILE, ROWS_PER_TILE)],
                    acc_hbm.at[c, pl.ds(s * ROWS_PER_TILE, ROWS_PER_TILE)])


_scatter = pl.kernel(
    _sc_scatter,
    out_type=jax.ShapeDtypeStruct((NC, NPAD, D), jnp.float32),
    mesh=_MESH,
    compiler_params=_SC_PARAMS,
    scratch_types=[
        pltpu.VMEM((PASS_CH, CH), jnp.int32),
        pltpu.VMEM((PASS_CH, CH), jnp.int32),
        pltpu.VMEM((NBUF, CH, D), jnp.float32),
        pltpu.VMEM_SHARED((NPAD, D), jnp.float32),
        pltpu.SemaphoreType.DMA,
        pltpu.SemaphoreType.DMA,
        pltpu.SemaphoreType.DMA,
        pltpu.SemaphoreType.DMA,
        pltpu.SemaphoreType.DMA,
        pltpu.SemaphoreType.DMA,
        pltpu.SemaphoreType.DMA,
        pltpu.SemaphoreType.DMA,
    ],
)


BLK_F = 512


def _tc_finalize(acc_ref, deg_ref, b_ref, o_ref):
    inv = _row_scale_col(deg_ref[0, :, 1, :])
    o_ref[...] = inv * (acc_ref[0] + acc_ref[1]) + b_ref[...]


_finalize = pl.pallas_call(
    _tc_finalize,
    grid=(NPAD // BLK_F,),
    in_specs=[
        pl.BlockSpec((NC, BLK_F, D), lambda i: (0, i, 0)),
        pl.BlockSpec((1, NW, 2, BLK_F), lambda i: (0, 0, 0, i)),
        pl.BlockSpec((1, D), lambda i: (0, 0)),
    ],
    out_specs=pl.BlockSpec((BLK_F, D), lambda i: (i, 0)),
    out_shape=jax.ShapeDtypeStruct((N, D), jnp.float32),
)


def kernel(x, edge_index, W_src, b_src, W_dst, b_dst):
    del W_dst, b_dst
    pad = EPAD - E
    sent = N + (jnp.arange(pad, dtype=jnp.int32) % (NPAD - N))
    row_p = jnp.concatenate([edge_index[0], sent]).reshape(NW * CPT, CH)
    col_p = jnp.concatenate([edge_index[1], sent]).reshape(NW * CPT, CH)
    x_p = jnp.pad(x, ((0, NPAD - N), (0, 0)))

    zflat = jnp.zeros((2 * NPAD,), jnp.float32)
    zrows = jnp.zeros((ROWS_PER_TILE, D), jnp.float32)

    deg = _degrees(row_p, col_p, zflat)
    deg4 = deg[None]
    z = _project(x_p, W_src, deg4)
    acc = _scatter(row_p, col_p, z, zrows)
    return _finalize(acc, deg4, b_src.reshape(1, D))
